# speedup vs baseline: 9.1437x; 9.1437x over previous
import functools
import jax
import jax.numpy as jnp
from jax import lax
from jax.experimental import pallas as pl
from jax.experimental.pallas import tpu as pltpu
from jax.experimental.pallas import tpu_sc as plsc

VOCAB = 100000
EMB = 128

NC = 2
NS = 16
NW = NC * NS

CHUNK = 128
NBUF = 4


def _make_kernel(n_rows: int):
    assert n_rows % (NW * CHUNK) == 0
    rows_per_w = n_rows // NW
    n_chunks = rows_per_w // CHUNK
    assert n_chunks % NBUF == 0
    n_groups = n_chunks // NBUF

    mesh = plsc.VectorSubcoreMesh(core_axis_name="c", subcore_axis_name="s")

    scratch = (
        [pltpu.VMEM((n_chunks, CHUNK), jnp.int32)]
        + [pltpu.VMEM((CHUNK, EMB), jnp.float32) for _ in range(NBUF)]
        + [pltpu.SemaphoreType.DMA for _ in range(2 * NBUF)]
    )

    @functools.partial(
        pl.kernel,
        out_type=jax.ShapeDtypeStruct((n_rows, EMB), jnp.float32),
        mesh=mesh,
        scratch_types=scratch,
    )
    def gather_kernel(idx_hbm, table_hbm, out_hbm, idx_v, *rest):
        bufs = rest[:NBUF]
        gsem = rest[NBUF : 2 * NBUF]
        ssem = rest[2 * NBUF : 3 * NBUF]

        wid = lax.axis_index("s") * NC + lax.axis_index("c")
        row_base = wid * rows_per_w

        pltpu.sync_copy(idx_hbm.at[wid], idx_v)

        def start_gather(b, c):
            return pltpu.async_copy(table_hbm.at[idx_v.at[c]], bufs[b], gsem[b])

        def start_scatter(b, c):
            dst = out_hbm.at[pl.ds(row_base + c * CHUNK, CHUNK)]
            return pltpu.async_copy(bufs[b], dst, ssem[b])

        def wait_gather(b):
            pltpu.make_async_copy(table_hbm.at[idx_v.at[0]], bufs[b], gsem[b]).wait()

        def wait_scatter(b):
            dst = out_hbm.at[pl.ds(row_base, CHUNK)]
            pltpu.make_async_copy(bufs[b], dst, ssem[b]).wait()

        for b in range(NBUF):
            start_gather(b, b)

        def group_body(g, carry):
            for b in range(NBUF):
                c = g * NBUF + b
                wait_gather(b)
                start_scatter(b, c)
            @pl.when(g + 1 < n_groups)
            def _():
                for b in range(NBUF):
                    c = (g + 1) * NBUF + b
                    wait_scatter(b)
                    start_gather(b, c)

            return carry

        lax.fori_loop(0, n_groups, group_body, 0)

        for b in range(NBUF):
            wait_scatter(b)

    return gather_kernel


@jax.jit
def kernel(input_seqs, input_lengths, table):
    del input_lengths
    n_rows = input_seqs.shape[0] * input_seqs.shape[1]
    idx3 = input_seqs.reshape(NW, n_rows // (NW * CHUNK), CHUNK)
    out = _make_kernel(n_rows)(idx3, table)
    return out.reshape(input_seqs.shape[0], input_seqs.shape[1], EMB)


# speedup vs baseline: 9.2440x; 1.0110x over previous
import functools
import jax
import jax.numpy as jnp
from jax import lax
from jax.experimental import pallas as pl
from jax.experimental.pallas import tpu as pltpu
from jax.experimental.pallas import tpu_sc as plsc

VOCAB = 100000
EMB = 128

NC = 2
NS = 16
NW = NC * NS

CHUNK = 128
NBUF = 5


def _make_kernel(n_rows: int):
    assert n_rows % (NW * CHUNK) == 0
    rows_per_w = n_rows // NW
    n_chunks = rows_per_w // CHUNK
    assert n_chunks % NBUF == 0
    n_groups = n_chunks // NBUF

    mesh = plsc.VectorSubcoreMesh(core_axis_name="c", subcore_axis_name="s")

    scratch = (
        [pltpu.VMEM((n_chunks, CHUNK), jnp.int32)]
        + [pltpu.VMEM((CHUNK, EMB), jnp.float32) for _ in range(NBUF)]
        + [pltpu.SemaphoreType.DMA for _ in range(2 * NBUF)]
    )

    @functools.partial(
        pl.kernel,
        out_type=jax.ShapeDtypeStruct((n_rows, EMB), jnp.float32),
        mesh=mesh,
        scratch_types=scratch,
    )
    def gather_kernel(idx_hbm, table_hbm, out_hbm, idx_v, *rest):
        bufs = rest[:NBUF]
        gsem = rest[NBUF : 2 * NBUF]
        ssem = rest[2 * NBUF : 3 * NBUF]

        wid = lax.axis_index("s") * NC + lax.axis_index("c")
        row_base = wid * rows_per_w

        pltpu.sync_copy(idx_hbm.at[wid], idx_v)

        def start_gather(b, c):
            return pltpu.async_copy(table_hbm.at[idx_v.at[c]], bufs[b], gsem[b])

        def start_scatter(b, c):
            dst = out_hbm.at[pl.ds(row_base + c * CHUNK, CHUNK)]
            return pltpu.async_copy(bufs[b], dst, ssem[b])

        def wait_gather(b):
            pltpu.make_async_copy(table_hbm.at[idx_v.at[0]], bufs[b], gsem[b]).wait()

        def wait_scatter(b):
            dst = out_hbm.at[pl.ds(row_base, CHUNK)]
            pltpu.make_async_copy(bufs[b], dst, ssem[b]).wait()

        for b in range(NBUF - 1):
            start_gather(b, b)

        def group_body(g, carry):
            for b in range(NBUF):
                c = g * NBUF + b
                q = (b - 1) % NBUF
                wait_gather(b)
                start_scatter(b, c)

                @pl.when(c > 0)
                def _():
                    wait_scatter(q)

                @pl.when(c + NBUF - 1 < n_chunks)
                def _():
                    start_gather(q, c + NBUF - 1)

            return carry

        lax.fori_loop(0, n_groups, group_body, 0)

        wait_scatter((n_chunks - 1) % NBUF)

    return gather_kernel


@jax.jit
def kernel(input_seqs, input_lengths, table):
    del input_lengths
    n_rows = input_seqs.shape[0] * input_seqs.shape[1]
    idx3 = input_seqs.reshape(NW, n_rows // (NW * CHUNK), CHUNK)
    out = _make_kernel(n_rows)(idx3, table)
    return out.reshape(input_seqs.shape[0], input_seqs.shape[1], EMB)


# speedup vs baseline: 9.2734x; 1.0032x over previous
import functools
import jax
import jax.numpy as jnp
from jax import lax
from jax.experimental import pallas as pl
from jax.experimental.pallas import tpu as pltpu
from jax.experimental.pallas import tpu_sc as plsc

VOCAB = 100000
EMB = 128

NC = 2
NS = 16
NW = NC * NS

CHUNK = 128
NBUF = 7


def _make_kernel(n_rows: int):
    assert n_rows % (NW * CHUNK) == 0
    rows_per_w = n_rows // NW
    n_chunks = rows_per_w // CHUNK
    n_groups = n_chunks // NBUF
    n_tail = n_chunks - n_groups * NBUF

    mesh = plsc.VectorSubcoreMesh(core_axis_name="c", subcore_axis_name="s")

    scratch = (
        [pltpu.VMEM((NBUF, CHUNK), jnp.int32)]
        + [pltpu.VMEM((CHUNK, EMB), jnp.float32) for _ in range(NBUF)]
        + [pltpu.SemaphoreType.DMA for _ in range(3 * NBUF)]
    )

    @functools.partial(
        pl.kernel,
        out_type=jax.ShapeDtypeStruct((n_rows, EMB), jnp.float32),
        mesh=mesh,
        scratch_types=scratch,
    )
    def gather_kernel(idx_hbm, table_hbm, out_hbm, idx_v, *rest):
        bufs = rest[:NBUF]
        gsem = rest[NBUF : 2 * NBUF]
        ssem = rest[2 * NBUF : 3 * NBUF]
        isem = rest[3 * NBUF : 4 * NBUF]

        wid = lax.axis_index("s") * NC + lax.axis_index("c")
        row_base = wid * rows_per_w

        def start_idx_load(b, c):
            pltpu.async_copy(idx_hbm.at[wid, c], idx_v.at[b], isem[b])

        def wait_idx_load(b):
            pltpu.make_async_copy(
                idx_hbm.at[wid, 0], idx_v.at[b], isem[b]
            ).wait()

        def start_gather(b, c):
            del c
            pltpu.async_copy(table_hbm.at[idx_v.at[b]], bufs[b], gsem[b])

        def start_scatter(b, c):
            dst = out_hbm.at[pl.ds(row_base + c * CHUNK, CHUNK)]
            pltpu.async_copy(bufs[b], dst, ssem[b])

        def wait_gather(b):
            pltpu.make_async_copy(table_hbm.at[idx_v.at[0]], bufs[b], gsem[b]).wait()

        def wait_scatter(b):
            dst = out_hbm.at[pl.ds(row_base, CHUNK)]
            pltpu.make_async_copy(bufs[b], dst, ssem[b]).wait()

        for b in range(NBUF):
            start_idx_load(b, b)
        for b in range(NBUF - 1):
            wait_idx_load(b)
            start_gather(b, b)

        def step(c, b, q,
                 has_prev_scatter=None, has_next_gather=None,
                 has_idx_prefetch=None):
            def do_all():
                wait_gather(b)

                def idx_prefetch():
                    start_idx_load(b, c + NBUF)

                if has_idx_prefetch is None:
                    pl.when(c + NBUF < n_chunks)(idx_prefetch)
                elif has_idx_prefetch:
                    idx_prefetch()

                start_scatter(b, c)

                def prev_scatter():
                    wait_scatter(q)

                if has_prev_scatter is None:
                    pl.when(c > 0)(prev_scatter)
                elif has_prev_scatter:
                    prev_scatter()

                def next_gather():
                    wait_idx_load(q)
                    start_gather(q, c + NBUF - 1)

                if has_next_gather is None:
                    pl.when(c + NBUF - 1 < n_chunks)(next_gather)
                elif has_next_gather:
                    next_gather()

            do_all()

        def group_body(g, carry):
            for b in range(NBUF):
                c = g * NBUF + b
                step(c, b, (b - 1) % NBUF)
            return carry

        lax.fori_loop(0, n_groups, group_body, 0)

        for t in range(n_tail):
            c = n_groups * NBUF + t
            step(
                c,
                c % NBUF,
                (c - 1) % NBUF,
                has_prev_scatter=True,
                has_next_gather=(c + NBUF - 1 < n_chunks),
                has_idx_prefetch=(c + NBUF < n_chunks),
            )

        wait_scatter((n_chunks - 1) % NBUF)

    return gather_kernel


@jax.jit
def kernel(input_seqs, input_lengths, table):
    del input_lengths
    n_rows = input_seqs.shape[0] * input_seqs.shape[1]
    idx3 = input_seqs.reshape(NW, n_rows // (NW * CHUNK), CHUNK)
    out = _make_kernel(n_rows)(idx3, table)
    return out.reshape(input_seqs.shape[0], input_seqs.shape[1], EMB)


# speedup vs baseline: 9.7092x; 1.0470x over previous
import functools
import jax
import jax.numpy as jnp
from jax import lax
from jax.experimental import pallas as pl
from jax.experimental.pallas import tpu as pltpu
from jax.experimental.pallas import tpu_sc as plsc

VOCAB = 100000
EMB = 128

NC = 2
NS = 16
NW = NC * NS

CHUNK = 64
NBUF = 6


def _make_kernel(n_rows: int):
    assert n_rows % (NW * CHUNK) == 0
    rows_per_w = n_rows // NW
    n_chunks = rows_per_w // CHUNK
    n_groups = n_chunks // NBUF
    n_tail = n_chunks - n_groups * NBUF
    assert n_chunks > 2 * NBUF

    mesh = plsc.VectorSubcoreMesh(core_axis_name="c", subcore_axis_name="s")

    scratch = (
        [
            pltpu.VMEM((NBUF, CHUNK), jnp.int32),
            pltpu.VMEM_SHARED((NS, NBUF, CHUNK, EMB), jnp.float32),
        ]
        + [pltpu.VMEM((CHUNK, EMB), jnp.float32) for _ in range(NBUF)]
        + [pltpu.SemaphoreType.DMA for _ in range(4 * NBUF)]
    )

    @functools.partial(
        pl.kernel,
        out_type=jax.ShapeDtypeStruct((n_rows, EMB), jnp.float32),
        mesh=mesh,
        scratch_types=scratch,
    )
    def gather_kernel(idx_hbm, table_hbm, out_hbm, idx_v, spmem, *rest):
        bufs = rest[:NBUF]
        sems = rest[NBUF:]
        gsem = sems[:NBUF]
        isem = sems[NBUF : 2 * NBUF]
        xsem = sems[2 * NBUF : 3 * NBUF]
        wsem = sems[3 * NBUF : 4 * NBUF]

        wid = lax.axis_index("s") * NC + lax.axis_index("c")
        sid = lax.axis_index("s")
        row_base = wid * rows_per_w

        def start_idx_load(b, c):
            pltpu.async_copy(idx_hbm.at[wid, c], idx_v.at[b], isem[b])

        def wait_idx_load(b):
            pltpu.make_async_copy(
                idx_hbm.at[wid, 0], idx_v.at[b], isem[b]
            ).wait()

        def start_gather(b):
            pltpu.async_copy(table_hbm.at[idx_v.at[b]], bufs[b], gsem[b])

        def wait_gather(b):
            pltpu.make_async_copy(
                table_hbm.at[idx_v.at[0]], bufs[b], gsem[b]
            ).wait()

        def start_x(b):
            pltpu.async_copy(bufs[b], spmem.at[sid, b], xsem[b])

        def wait_x(b):
            pltpu.make_async_copy(bufs[0], spmem.at[sid, b], xsem[b]).wait()

        def start_w(b, c):
            dst = out_hbm.at[pl.ds(row_base + c * CHUNK, CHUNK)]
            pltpu.async_copy(spmem.at[sid, b], dst, wsem[b])

        def wait_w(b):
            dst = out_hbm.at[pl.ds(row_base, CHUNK)]
            pltpu.make_async_copy(spmem.at[sid, 0], dst, wsem[b]).wait()

        for b in range(NBUF):
            start_idx_load(b, b)
        for b in range(NBUF - 1):
            wait_idx_load(b)
            start_gather(b)

        def step(c, b, q, static=False):
            def guarded(cond, fn):
                if static:
                    if cond:
                        fn()
                else:
                    pl.when(cond)(fn)

            wait_gather(b)
            guarded(c + NBUF < n_chunks, lambda: start_idx_load(b, c + NBUF))
            guarded(c >= NBUF, lambda: wait_w(b))
            start_x(b)

            def after_prev_x():
                wait_x(q)
                start_w(q, c - 1)

            guarded(c >= 1, after_prev_x)

            def next_gather():
                wait_idx_load(q)
                start_gather(q)

            guarded(c + NBUF - 1 < n_chunks, next_gather)

        def group_body(g, carry):
            for b in range(NBUF):
                step(g * NBUF + b, b, (b - 1) % NBUF)
            return carry

        lax.fori_loop(0, n_groups, group_body, 0)

        for t in range(n_tail):
            c = n_groups * NBUF + t
            step(c, c % NBUF, (c - 1) % NBUF, static=True)

        last = n_chunks - 1
        wait_x(last % NBUF)
        start_w(last % NBUF, last)
        for k in range(NBUF):
            wait_w((n_chunks - NBUF + k) % NBUF)

    return gather_kernel


@jax.jit
def kernel(input_seqs, input_lengths, table):
    del input_lengths
    n_rows = input_seqs.shape[0] * input_seqs.shape[1]
    idx3 = input_seqs.reshape(NW, n_rows // (NW * CHUNK), CHUNK)
    out = _make_kernel(n_rows)(idx3, table)
    return out.reshape(input_seqs.shape[0], input_seqs.shape[1], EMB)


# speedup vs baseline: 9.7620x; 1.0054x over previous
import functools
import jax
import jax.numpy as jnp
from jax import lax
from jax.experimental import pallas as pl
from jax.experimental.pallas import tpu as pltpu
from jax.experimental.pallas import tpu_sc as plsc

VOCAB = 100000
EMB = 128

NC = 2
NS = 16
NW = NC * NS

CHUNK = 64
NBUF = 6
LOOK = NBUF - 1
IB = 8
NIB = 3
SPAN = 24


def _make_kernel(n_rows: int):
    assert n_rows % (NW * CHUNK) == 0
    rows_per_w = n_rows // NW
    n_chunks = rows_per_w // CHUNK
    assert n_chunks % IB == 0, "index batches must tile the chunk axis"
    n_super = n_chunks // SPAN
    n_tail = n_chunks - n_super * SPAN
    assert n_chunks > 2 * SPAN

    mesh = plsc.VectorSubcoreMesh(core_axis_name="c", subcore_axis_name="s")

    scratch = (
        [
            pltpu.VMEM((NIB, IB, CHUNK), jnp.int32),
            pltpu.VMEM_SHARED((NS, NBUF, CHUNK, EMB), jnp.float32),
        ]
        + [pltpu.VMEM((CHUNK, EMB), jnp.float32) for _ in range(NBUF)]
        + [pltpu.SemaphoreType.DMA for _ in range(3 * NBUF + NIB)]
    )

    @functools.partial(
        pl.kernel,
        out_type=jax.ShapeDtypeStruct((n_rows, EMB), jnp.float32),
        mesh=mesh,
        scratch_types=scratch,
    )
    def gather_kernel(idx_hbm, table_hbm, out_hbm, idx_v, spmem, *rest):
        bufs = rest[:NBUF]
        sems = rest[NBUF:]
        gsem = sems[:NBUF]
        xsem = sems[NBUF : 2 * NBUF]
        wsem = sems[2 * NBUF : 3 * NBUF]
        isem = sems[3 * NBUF : 3 * NBUF + NIB]

        wid = lax.axis_index("s") * NC + lax.axis_index("c")
        sid = lax.axis_index("s")
        row_base = wid * rows_per_w

        def start_idx_batch(s, first_chunk):
            pltpu.async_copy(
                idx_hbm.at[wid, pl.ds(first_chunk, IB)], idx_v.at[s], isem[s]
            )

        def wait_idx_batch(s):
            pltpu.make_async_copy(
                idx_hbm.at[wid, pl.ds(0, IB)], idx_v.at[s], isem[s]
            ).wait()

        def start_gather(b, s, row):
            pltpu.async_copy(
                table_hbm.at[idx_v.at[s, row]], bufs[b], gsem[b]
            )

        def wait_gather(b):
            pltpu.make_async_copy(
                table_hbm.at[idx_v.at[0, 0]], bufs[b], gsem[b]
            ).wait()

        def start_x(b):
            pltpu.async_copy(bufs[b], spmem.at[sid, b], xsem[b])

        def wait_x(b):
            pltpu.make_async_copy(bufs[0], spmem.at[sid, b], xsem[b]).wait()

        def start_w(b, c):
            dst = out_hbm.at[pl.ds(row_base + c * CHUNK, CHUNK)]
            pltpu.async_copy(spmem.at[sid, b], dst, wsem[b])

        def wait_w(b):
            dst = out_hbm.at[pl.ds(row_base, CHUNK)]
            pltpu.make_async_copy(spmem.at[sid, 0], dst, wsem[b]).wait()

        for s in range(NIB):
            start_idx_batch(s, s * IB)
        wait_idx_batch(0)
        for b in range(LOOK):
            start_gather(b, 0, b)

        def step(c, b, gslot, grow, static=False):
            def guarded(cond, fn):
                if static:
                    if cond:
                        fn()
                else:
                    pl.when(cond)(fn)

            q = (b - 1) % NBUF
            wait_gather(b)
            guarded(c >= NBUF, lambda: wait_w(b))
            start_x(b)

            def after_prev_x():
                wait_x(q)
                start_w(q, c - 1)

            guarded(c >= 1, after_prev_x)
            guarded(c + LOOK < n_chunks,
                    lambda: start_gather(q, gslot, grow))

        def super_body(S, carry):
            for k in range(SPAN):
                c = S * SPAN + k
                x = k + LOOK
                gslot = (x // IB) % NIB
                grow = x % IB
                if k > 0 and grow == 0:
                    wait_idx_batch(gslot)
                step(c, k % NBUF, gslot, grow)
                if (k + 2 * IB + 1) % IB == 0:
                    load_slot = ((k + 2 * IB + 1) // IB) % NIB
                    pl.when(c + 3 * IB + 1 <= n_chunks)(
                        lambda: start_idx_batch(load_slot, c + 2 * IB + 1)
                    )
            return carry

        lax.fori_loop(0, n_super, super_body, 0)

        for t in range(n_tail):
            c = n_super * SPAN + t
            x = c + LOOK
            if x < n_chunks and x % IB == 0:
                wait_idx_batch((x // IB) % NIB)
            step(c, c % NBUF, (x // IB) % NIB, x % IB, static=True)

        last = n_chunks - 1
        wait_x(last % NBUF)
        start_w(last % NBUF, last)
        for k in range(NBUF):
            wait_w((n_chunks - NBUF + k) % NBUF)

    return gather_kernel


@jax.jit
def kernel(input_seqs, input_lengths, table):
    del input_lengths
    n_rows = input_seqs.shape[0] * input_seqs.shape[1]
    n_chunks = n_rows // (NW * CHUNK)
    idx3 = input_seqs.reshape(NW, n_chunks, CHUNK)
    out = _make_kernel(n_rows)(idx3, table)
    return out.reshape(input_seqs.shape[0], input_seqs.shape[1], EMB)
